# Initial kernel scaffold; baseline (speedup 1.0000x reference)
import functools

import jax
import jax.numpy as jnp
from jax import lax
from jax.experimental import pallas as pl
from jax.experimental.pallas import tpu as pltpu
from jax.experimental.pallas import tpu_sc as plsc

N = 10000
E = 320000
D = 128
H = 4
C = 32
B = 64

NC = 2
NS = 16
L = 16

KCH = 80
EPT = E // (NC * NS)
NCHUNK = EPT // KCH
ROWS_PT = 624
TAIL0 = NS * ROWS_PT
TAILN = N - TAIL0

_MESH = plsc.VectorSubcoreMesh(core_axis_name="c", subcore_axis_name="s")


def _sc_gat_body(src_hbm, dst_hbm, h_hbm, als_hbm, ald_hbm, z128_hbm, z8_hbm,
                 acc_out, den_out,
                 als_v, ald_v, srcb, dstb, rows, den_v, ebuf, acc_sh, den_sh, sem):
    cid = lax.axis_index("c")
    sid = lax.axis_index("s")
    wid = sid * NC + cid

    pltpu.sync_copy(als_hbm, als_v)
    pltpu.sync_copy(ald_hbm, ald_v)

    r0 = sid * ROWS_PT
    pltpu.sync_copy(z128_hbm.at[pl.ds(r0, ROWS_PT)], acc_sh.at[pl.ds(r0, ROWS_PT)])
    pltpu.sync_copy(z8_hbm.at[pl.ds(r0, ROWS_PT)], den_sh.at[pl.ds(r0, ROWS_PT)])

    @pl.when(sid == NS - 1)
    def _():
        pltpu.sync_copy(z128_hbm.at[pl.ds(TAIL0, TAILN)], acc_sh.at[pl.ds(TAIL0, TAILN)])
        pltpu.sync_copy(z8_hbm.at[pl.ds(TAIL0, TAILN)], den_sh.at[pl.ds(TAIL0, TAILN)])

    zero16 = jnp.zeros((L,), jnp.float32)
    for g in range(KCH // L):
        ridx = jnp.arange(L, dtype=jnp.int32) + (g * L)
        for col in (5, 6, 7):
            cidx = jnp.full((L,), col, jnp.int32)
            plsc.store_scatter(den_v, [ridx, cidx], zero16)

    plsc.subcore_barrier()

    ebase = wid * EPT

    def chunk_body(c, carry):
        base = ebase + c * KCH
        pltpu.sync_copy(src_hbm.at[pl.ds(base, KCH)], srcb)
        pltpu.sync_copy(dst_hbm.at[pl.ds(base, KCH)], dstb)
        pltpu.async_copy(h_hbm.at[srcb], rows, sem).wait()

        for g in range(KCH // L):
            s16 = srcb[pl.ds(g * L, L)]
            d16 = dstb[pl.ds(g * L, L)]
            ridx = jnp.arange(L, dtype=jnp.int32) + (g * L)
            ones16 = jnp.full((L,), 1.0, jnp.float32)
            plsc.store_scatter(den_v, [ridx, jnp.full((L,), 4, jnp.int32)], ones16)
            for hh in range(H):
                hidx = jnp.full((L,), hh, jnp.int32)
                a_s = plsc.load_gather(als_v, [s16, hidx])
                a_d = plsc.load_gather(ald_v, [d16, hidx])
                z = a_s + a_d
                ee = jnp.exp(jnp.maximum(z, 0.2 * z))
                ebuf[hh, pl.ds(g * L, L)] = ee
                plsc.store_scatter(den_v, [ridx, jnp.full((L,), hh, jnp.int32)], ee)

        for i in range(KCH):
            for hh in range(H):
                s = ebuf[hh, i]
                sv = jnp.full((L,), s)
                for half in range(2):
                    sl = pl.ds(hh * C + half * L, L)
                    rows[i, sl] = rows[i, sl] * sv

        pltpu.sync_copy(rows, acc_sh.at[dstb], add=True)
        pltpu.sync_copy(den_v, den_sh.at[dstb], add=True)
        return carry

    lax.fori_loop(0, NCHUNK, chunk_body, 0)

    plsc.subcore_barrier()

    pltpu.sync_copy(acc_sh.at[pl.ds(r0, ROWS_PT)], acc_out.at[cid, pl.ds(r0, ROWS_PT)])
    pltpu.sync_copy(den_sh.at[pl.ds(r0, ROWS_PT)], den_out.at[cid, pl.ds(r0, ROWS_PT)])

    @pl.when(sid == NS - 1)
    def _():
        pltpu.sync_copy(acc_sh.at[pl.ds(TAIL0, TAILN)], acc_out.at[cid, pl.ds(TAIL0, TAILN)])
        pltpu.sync_copy(den_sh.at[pl.ds(TAIL0, TAILN)], den_out.at[cid, pl.ds(TAIL0, TAILN)])


_sc_gat_pass = functools.partial(
    pl.kernel,
    _sc_gat_body,
    out_type=(
        jax.ShapeDtypeStruct((NC, N, D), jnp.float32),
        jax.ShapeDtypeStruct((NC, N, 8), jnp.float32),
    ),
    mesh=_MESH,
    scratch_types=[
        pltpu.VMEM((N, H), jnp.float32),
        pltpu.VMEM((N, H), jnp.float32),
        pltpu.VMEM((KCH,), jnp.int32),
        pltpu.VMEM((KCH,), jnp.int32),
        pltpu.VMEM((KCH, D), jnp.float32),
        pltpu.VMEM((KCH, 8), jnp.float32),
        pltpu.VMEM((H, KCH), jnp.float32),
        pltpu.VMEM_SHARED((N, D), jnp.float32),
        pltpu.VMEM_SHARED((N, 8), jnp.float32),
        pltpu.SemaphoreType.DMA,
    ],
)


def _sc_lp_body(src_hbm, dst_hbm, g_hbm, z128_hbm, agg_out,
                srcb, dstb, rows, acc_sh, sem):
    cid = lax.axis_index("c")
    sid = lax.axis_index("s")
    wid = sid * NC + cid

    r0 = sid * ROWS_PT
    pltpu.sync_copy(z128_hbm.at[pl.ds(r0, ROWS_PT)], acc_sh.at[pl.ds(r0, ROWS_PT)])

    @pl.when(sid == NS - 1)
    def _():
        pltpu.sync_copy(z128_hbm.at[pl.ds(TAIL0, TAILN)], acc_sh.at[pl.ds(TAIL0, TAILN)])

    plsc.subcore_barrier()

    ebase = wid * EPT

    def chunk_body(c, carry):
        base = ebase + c * KCH
        pltpu.sync_copy(src_hbm.at[pl.ds(base, KCH)], srcb)
        pltpu.sync_copy(dst_hbm.at[pl.ds(base, KCH)], dstb)
        pltpu.async_copy(g_hbm.at[srcb], rows, sem).wait()
        pltpu.sync_copy(rows, acc_sh.at[dstb], add=True)
        return carry

    lax.fori_loop(0, NCHUNK, chunk_body, 0)

    plsc.subcore_barrier()

    pltpu.sync_copy(acc_sh.at[pl.ds(r0, ROWS_PT)], agg_out.at[cid, pl.ds(r0, ROWS_PT)])

    @pl.when(sid == NS - 1)
    def _():
        pltpu.sync_copy(acc_sh.at[pl.ds(TAIL0, TAILN)], agg_out.at[cid, pl.ds(TAIL0, TAILN)])


_sc_lp_pass = functools.partial(
    pl.kernel,
    _sc_lp_body,
    out_type=jax.ShapeDtypeStruct((NC, N, D), jnp.float32),
    mesh=_MESH,
    scratch_types=[
        pltpu.VMEM((KCH,), jnp.int32),
        pltpu.VMEM((KCH,), jnp.int32),
        pltpu.VMEM((KCH, D), jnp.float32),
        pltpu.VMEM_SHARED((N, D), jnp.float32),
        pltpu.SemaphoreType.DMA,
    ],
)


def _tc_proj_body(x_ref, w_ref, as_ref, ad_ref, h_ref, als_ref, ald_ref):
    h = jnp.dot(x_ref[...], w_ref[...], preferred_element_type=jnp.float32)
    h_ref[...] = h
    als_ref[...] = jnp.dot(h, as_ref[...], preferred_element_type=jnp.float32)
    ald_ref[...] = jnp.dot(h, ad_ref[...], preferred_element_type=jnp.float32)


def _tc_proj(x, w, a_s, a_d):
    return pl.pallas_call(
        _tc_proj_body,
        out_shape=(
            jax.ShapeDtypeStruct((N, D), jnp.float32),
            jax.ShapeDtypeStruct((N, H), jnp.float32),
            jax.ShapeDtypeStruct((N, H), jnp.float32),
        ),
    )(x, w, a_s, a_d)


def _dis_from_den(den):
    deg = den[:, 4:5]
    return jnp.where(deg > 0, lax.rsqrt(jnp.maximum(deg, 1e-12)), 0.0)


def _tc_gatfin_body(accp_ref, denp_ref, b_ref, s_ref, h_ref, g_ref):
    acc = accp_ref[0] + accp_ref[1]
    den = denp_ref[0] + denp_ref[1]
    r = 1.0 / (den[:, 0:4] + 1e-16)
    r_rep = jnp.dot(r, s_ref[...], preferred_element_type=jnp.float32)
    h = jnp.maximum(acc * r_rep + b_ref[...], 0.0)
    h_ref[...] = h
    g_ref[...] = h * _dis_from_den(den)


def _tc_gatfin(accp, denp, b2d, sel):
    return pl.pallas_call(
        _tc_gatfin_body,
        out_shape=(
            jax.ShapeDtypeStruct((N, D), jnp.float32),
            jax.ShapeDtypeStruct((N, D), jnp.float32),
        ),
    )(accp, denp, b2d, sel)


def _tc_lpfin_body(aggp_ref, hres_ref, denp_ref, out_ref, g_ref):
    den = denp_ref[0] + denp_ref[1]
    dis = _dis_from_den(den)
    agg = aggp_ref[0] + aggp_ref[1]
    out = jnp.clip(0.5 * dis * agg + 0.5 * hres_ref[...], 0.0, 1.0)
    out_ref[...] = out
    g_ref[...] = out * dis


def _tc_lpfin(aggp, hres, denp):
    return pl.pallas_call(
        _tc_lpfin_body,
        out_shape=(
            jax.ShapeDtypeStruct((N, D), jnp.float32),
            jax.ShapeDtypeStruct((N, D), jnp.float32),
        ),
    )(aggp, hres, denp)


def _tc_final_body(x_ref, h1_ref, h2_ref, bt_ref, w1_ref, b1_ref, w2_ref, b2_ref,
                   out_ref):
    combined = jnp.concatenate([x_ref[...], h1_ref[...], h2_ref[...]], axis=-1)
    bt = bt_ref[...]
    oh = (lax.broadcasted_iota(jnp.int32, (B, N), 0) == bt).astype(jnp.float32)
    pooled_sum = jnp.dot(oh, combined, preferred_element_type=jnp.float32)
    counts = jnp.sum(oh, axis=1, keepdims=True)
    pooled = pooled_sum / jnp.maximum(counts, 1.0)
    hmid = jnp.maximum(
        jnp.dot(pooled, w1_ref[...], preferred_element_type=jnp.float32) + b1_ref[...],
        0.0)
    out_ref[...] = jnp.dot(hmid, w2_ref[...], preferred_element_type=jnp.float32) + b2_ref[...]


def _tc_final(x, h1, h2, bt, w1, b1, w2, b2):
    return pl.pallas_call(
        _tc_final_body,
        out_shape=jax.ShapeDtypeStruct((B, 128), jnp.float32),
    )(x, h1, h2, bt, w1, b1, w2, b2)


def kernel(x, edge_index, batch, W1, a1_src, a1_dst, b1, W2, a2_src, a2_dst, b2,
           mlp_w1, mlp_b1, mlp_w2, mlp_b2):
    src = edge_index[0]
    dst = edge_index[1]

    eye = jnp.eye(H, dtype=jnp.float32)
    As1 = jnp.einsum('hc,hk->hck', a1_src, eye).reshape(D, H)
    Ad1 = jnp.einsum('hc,hk->hck', a1_dst, eye).reshape(D, H)
    As2 = jnp.einsum('hc,hk->hck', a2_src, eye).reshape(D, H)
    Ad2 = jnp.einsum('hc,hk->hck', a2_dst, eye).reshape(D, H)
    sel = jnp.repeat(jnp.eye(H, dtype=jnp.float32), C, axis=1)

    z128 = jnp.zeros((N, D), jnp.float32)
    z8 = jnp.zeros((N, 8), jnp.float32)
    b1_2d = b1.reshape(1, D)
    b2_2d = b2.reshape(1, D)
    bt = batch.reshape(1, N)

    h1p, als1, ald1 = _tc_proj(x, W1, As1, Ad1)
    accp1, denp1 = _sc_gat_pass(src, dst, h1p, als1, ald1, z128, z8)
    h1, g = _tc_gatfin(accp1, denp1, b1_2d, sel)
    aggp = _sc_lp_pass(src, dst, g, z128)
    _, g = _tc_lpfin(aggp, h1, denp1)
    aggp = _sc_lp_pass(src, dst, g, z128)
    h1f, _ = _tc_lpfin(aggp, h1, denp1)

    h2p, als2, ald2 = _tc_proj(h1f, W2, As2, Ad2)
    accp2, denp2 = _sc_gat_pass(src, dst, h2p, als2, ald2, z128, z8)
    h2, g = _tc_gatfin(accp2, denp2, b2_2d, sel)
    aggp = _sc_lp_pass(src, dst, g, z128)
    _, g = _tc_lpfin(aggp, h2, denp1)
    aggp = _sc_lp_pass(src, dst, g, z128)
    h2f, _ = _tc_lpfin(aggp, h2, denp1)

    return _tc_final(x, h1f, h2f, bt, mlp_w1, mlp_b1.reshape(1, 256),
                     mlp_w2, mlp_b2.reshape(1, 128))


# speedup vs baseline: 32.2497x; 32.2497x over previous
import functools

import jax
import jax.numpy as jnp
from jax import lax
from jax.experimental import pallas as pl
from jax.experimental.pallas import tpu as pltpu
from jax.experimental.pallas import tpu_sc as plsc

N = 10000
E = 320000
D = 128
H = 4
C = 32
B = 64

NC = 2
NS = 16
L = 16

KCH = 80
EPT = E // (NC * NS)
NCHUNK = EPT // KCH
ROWS_PT = 624
TAIL0 = NS * ROWS_PT
TAILN = N - TAIL0

_MESH = plsc.VectorSubcoreMesh(core_axis_name="c", subcore_axis_name="s")


DENW = 40960
DEN_PT = DENW // NS
DEGW = 10240
DEG_PT = DEGW // NS


def _sc_attn_body(src_hbm, dst_hbm, als_hbm, ald_hbm, z4_hbm, z1_hbm,
                  den_out, deg_out, ee_out,
                  als_v, ald_v, srcb, dstb, ee_st, idx0, idx1, idx2, idx3,
                  ones_v, den_sh, deg_sh, sem):
    cid = lax.axis_index("c")
    sid = lax.axis_index("s")
    wid = sid * NC + cid

    pltpu.sync_copy(als_hbm, als_v)
    pltpu.sync_copy(ald_hbm, ald_v)

    d0 = pl.multiple_of(sid * DEN_PT, 128)
    pltpu.sync_copy(z4_hbm.at[pl.ds(d0, DEN_PT)], den_sh.at[pl.ds(d0, DEN_PT)])
    r0 = pl.multiple_of(sid * DEG_PT, 128)
    pltpu.sync_copy(z1_hbm.at[pl.ds(r0, DEG_PT)], deg_sh.at[pl.ds(r0, DEG_PT)])

    ones16 = jnp.full((L,), 1.0, jnp.float32)
    for g in range(KCH // L):
        ones_v[pl.ds(g * L, L)] = ones16

    plsc.subcore_barrier()

    ebase = wid * EPT
    idxs = (idx0, idx1, idx2, idx3)

    def chunk_body(c, carry):
        base = pl.multiple_of(ebase + c * KCH, 16)
        pltpu.sync_copy(src_hbm.at[pl.ds(base, KCH)], srcb)
        pltpu.sync_copy(dst_hbm.at[pl.ds(base, KCH)], dstb)

        for g in range(KCH // L):
            s16 = srcb[pl.ds(g * L, L)]
            d16 = dstb[pl.ds(g * L, L)]
            s4 = s16 * H
            d4 = d16 * H
            for hh in range(H):
                a_s = plsc.load_gather(als_v, [s4 + hh])
                a_d = plsc.load_gather(ald_v, [d4 + hh])
                z = a_s + a_d
                ee = jnp.exp(jnp.maximum(z, 0.2 * z))
                ee_st[pl.ds(hh * KCH + g * L, L)] = ee
                idxs[hh][pl.ds(g * L, L)] = d4 + hh

        for hh in range(H):
            pltpu.sync_copy(ee_st.at[pl.ds(hh * KCH, KCH)],
                            den_sh.at[idxs[hh]], add=True)
        pltpu.sync_copy(ones_v, deg_sh.at[dstb], add=True)
        pltpu.sync_copy(ee_st, ee_out.at[pl.ds(pl.multiple_of(base * H, 64), KCH * H)])
        return carry

    lax.fori_loop(0, NCHUNK, chunk_body, 0)

    plsc.subcore_barrier()

    pltpu.sync_copy(den_sh.at[pl.ds(d0, DEN_PT)], den_out.at[cid, 0, pl.ds(d0, DEN_PT)])
    pltpu.sync_copy(deg_sh.at[pl.ds(r0, DEG_PT)], deg_out.at[cid, 0, pl.ds(r0, DEG_PT)])


_sc_attn_pass = pl.kernel(
    _sc_attn_body,
    out_type=(
        jax.ShapeDtypeStruct((NC, 1, DENW), jnp.float32),
        jax.ShapeDtypeStruct((NC, 1, DEGW), jnp.float32),
        jax.ShapeDtypeStruct((E * H,), jnp.float32),
    ),
    mesh=_MESH,
    scratch_types=[
        pltpu.VMEM((N * H,), jnp.float32),
        pltpu.VMEM((N * H,), jnp.float32),
        pltpu.VMEM((KCH,), jnp.int32),
        pltpu.VMEM((KCH,), jnp.int32),
        pltpu.VMEM((KCH * H,), jnp.float32),
        pltpu.VMEM((KCH,), jnp.int32),
        pltpu.VMEM((KCH,), jnp.int32),
        pltpu.VMEM((KCH,), jnp.int32),
        pltpu.VMEM((KCH,), jnp.int32),
        pltpu.VMEM((KCH,), jnp.float32),
        pltpu.VMEM_SHARED((DENW,), jnp.float32),
        pltpu.VMEM_SHARED((DEGW,), jnp.float32),
        pltpu.SemaphoreType.DMA,
    ],
    compiler_params=pltpu.CompilerParams(needs_layout_passes=False),
)


def _sc_msg_body(src_hbm, dst_hbm, h_hbm, ee_hbm, z128_hbm,
                 acc_out,
                 srcb, dstb, rows, ee_v, acc_sh, sem):
    cid = lax.axis_index("c")
    sid = lax.axis_index("s")
    wid = sid * NC + cid

    r0 = sid * ROWS_PT
    pltpu.sync_copy(z128_hbm.at[pl.ds(r0, ROWS_PT)], acc_sh.at[pl.ds(r0, ROWS_PT)])

    @pl.when(sid == NS - 1)
    def _():
        pltpu.sync_copy(z128_hbm.at[pl.ds(TAIL0, TAILN)], acc_sh.at[pl.ds(TAIL0, TAILN)])

    plsc.subcore_barrier()

    ebase = wid * EPT

    def chunk_body(c, carry):
        base = ebase + c * KCH
        pltpu.sync_copy(src_hbm.at[pl.ds(base, KCH)], srcb)
        pltpu.sync_copy(dst_hbm.at[pl.ds(base, KCH)], dstb)
        pltpu.sync_copy(ee_hbm.at[pl.ds(pl.multiple_of(base * H, 64), KCH * H)], ee_v)
        pltpu.async_copy(h_hbm.at[srcb], rows, sem).wait()

        evs = [ee_v[pl.ds(v * L, L)] for v in range(KCH * H // L)]
        for i in range(KCH):
            for hh in range(H):
                j = hh * KCH + i
                sv = jnp.full((L,), evs[j // L][j % L])
                for half in range(2):
                    sl = pl.ds(hh * C + half * L, L)
                    rows[i, sl] = rows[i, sl] * sv

        pltpu.sync_copy(rows, acc_sh.at[dstb], add=True)
        return carry

    lax.fori_loop(0, NCHUNK, chunk_body, 0)

    plsc.subcore_barrier()

    pltpu.sync_copy(acc_sh.at[pl.ds(r0, ROWS_PT)], acc_out.at[cid, pl.ds(r0, ROWS_PT)])

    @pl.when(sid == NS - 1)
    def _():
        pltpu.sync_copy(acc_sh.at[pl.ds(TAIL0, TAILN)], acc_out.at[cid, pl.ds(TAIL0, TAILN)])


_sc_msg_pass = pl.kernel(
    _sc_msg_body,
    out_type=jax.ShapeDtypeStruct((NC, N, D), jnp.float32),
    mesh=_MESH,
    scratch_types=[
        pltpu.VMEM((KCH,), jnp.int32),
        pltpu.VMEM((KCH,), jnp.int32),
        pltpu.VMEM((KCH, D), jnp.float32),
        pltpu.VMEM((KCH * H,), jnp.float32),
        pltpu.VMEM_SHARED((N, D), jnp.float32),
        pltpu.SemaphoreType.DMA,
    ],
    compiler_params=pltpu.CompilerParams(needs_layout_passes=False),
)


def _sc_lp_body(src_hbm, dst_hbm, g_hbm, z128_hbm, agg_out,
                srcb, dstb, rows, acc_sh, sem):
    cid = lax.axis_index("c")
    sid = lax.axis_index("s")
    wid = sid * NC + cid

    r0 = sid * ROWS_PT
    pltpu.sync_copy(z128_hbm.at[pl.ds(r0, ROWS_PT)], acc_sh.at[pl.ds(r0, ROWS_PT)])

    @pl.when(sid == NS - 1)
    def _():
        pltpu.sync_copy(z128_hbm.at[pl.ds(TAIL0, TAILN)], acc_sh.at[pl.ds(TAIL0, TAILN)])

    plsc.subcore_barrier()

    ebase = wid * EPT

    def chunk_body(c, carry):
        base = ebase + c * KCH
        pltpu.sync_copy(src_hbm.at[pl.ds(base, KCH)], srcb)
        pltpu.sync_copy(dst_hbm.at[pl.ds(base, KCH)], dstb)
        pltpu.async_copy(g_hbm.at[srcb], rows, sem).wait()
        pltpu.sync_copy(rows, acc_sh.at[dstb], add=True)
        return carry

    lax.fori_loop(0, NCHUNK, chunk_body, 0)

    plsc.subcore_barrier()

    pltpu.sync_copy(acc_sh.at[pl.ds(r0, ROWS_PT)], agg_out.at[cid, pl.ds(r0, ROWS_PT)])

    @pl.when(sid == NS - 1)
    def _():
        pltpu.sync_copy(acc_sh.at[pl.ds(TAIL0, TAILN)], agg_out.at[cid, pl.ds(TAIL0, TAILN)])


_sc_lp_pass = pl.kernel(
    _sc_lp_body,
    out_type=jax.ShapeDtypeStruct((NC, N, D), jnp.float32),
    mesh=_MESH,
    scratch_types=[
        pltpu.VMEM((KCH,), jnp.int32),
        pltpu.VMEM((KCH,), jnp.int32),
        pltpu.VMEM((KCH, D), jnp.float32),
        pltpu.VMEM_SHARED((N, D), jnp.float32),
        pltpu.SemaphoreType.DMA,
    ],
    compiler_params=pltpu.CompilerParams(needs_layout_passes=False),
)


def _tc_proj_body(x_ref, w_ref, as_ref, ad_ref, h_ref, als_ref, ald_ref):
    h = jnp.dot(x_ref[...], w_ref[...], preferred_element_type=jnp.float32)
    h_ref[...] = h
    als_ref[...] = jnp.dot(h, as_ref[...], preferred_element_type=jnp.float32)
    ald_ref[...] = jnp.dot(h, ad_ref[...], preferred_element_type=jnp.float32)


def _tc_proj(x, w, a_s, a_d):
    return pl.pallas_call(
        _tc_proj_body,
        out_shape=(
            jax.ShapeDtypeStruct((N, D), jnp.float32),
            jax.ShapeDtypeStruct((N, H), jnp.float32),
            jax.ShapeDtypeStruct((N, H), jnp.float32),
        ),
    )(x, w, a_s, a_d)


def _dis_from_deg(degp):
    deg = degp[0] + degp[1]
    return jnp.where(deg > 0, lax.rsqrt(jnp.maximum(deg, 1e-12)), 0.0)


def _tc_gatfin_body(accp_ref, denp_ref, degp_ref, b_ref, s_ref, h_ref, g_ref):
    acc = accp_ref[0] + accp_ref[1]
    den = denp_ref[0] + denp_ref[1]
    r = 1.0 / (den + 1e-16)
    r_rep = jnp.dot(r, s_ref[...], preferred_element_type=jnp.float32)
    h = jnp.maximum(acc * r_rep + b_ref[...], 0.0)
    h_ref[...] = h
    g_ref[...] = h * _dis_from_deg(degp_ref)


def _tc_gatfin(accp, denp, degp, b2d, sel):
    return pl.pallas_call(
        _tc_gatfin_body,
        out_shape=(
            jax.ShapeDtypeStruct((N, D), jnp.float32),
            jax.ShapeDtypeStruct((N, D), jnp.float32),
        ),
    )(accp, denp, degp, b2d, sel)


def _tc_lpfin_body(aggp_ref, hres_ref, degp_ref, out_ref, g_ref):
    dis = _dis_from_deg(degp_ref)
    agg = aggp_ref[0] + aggp_ref[1]
    out = jnp.clip(0.5 * dis * agg + 0.5 * hres_ref[...], 0.0, 1.0)
    out_ref[...] = out
    g_ref[...] = out * dis


def _tc_lpfin(aggp, hres, degp):
    return pl.pallas_call(
        _tc_lpfin_body,
        out_shape=(
            jax.ShapeDtypeStruct((N, D), jnp.float32),
            jax.ShapeDtypeStruct((N, D), jnp.float32),
        ),
    )(aggp, hres, degp)


def _tc_final_body(x_ref, h1_ref, h2_ref, bt_ref, w1_ref, b1_ref, w2_ref, b2_ref,
                   out_ref):
    combined = jnp.concatenate([x_ref[...], h1_ref[...], h2_ref[...]], axis=-1)
    bt = bt_ref[...]
    oh = (lax.broadcasted_iota(jnp.int32, (B, N), 0) == bt).astype(jnp.float32)
    pooled_sum = jnp.dot(oh, combined, preferred_element_type=jnp.float32)
    counts = jnp.sum(oh, axis=1, keepdims=True)
    pooled = pooled_sum / jnp.maximum(counts, 1.0)
    hmid = jnp.maximum(
        jnp.dot(pooled, w1_ref[...], preferred_element_type=jnp.float32) + b1_ref[...],
        0.0)
    out_ref[...] = jnp.dot(hmid, w2_ref[...], preferred_element_type=jnp.float32) + b2_ref[...]


def _tc_final(x, h1, h2, bt, w1, b1, w2, b2):
    return pl.pallas_call(
        _tc_final_body,
        out_shape=jax.ShapeDtypeStruct((B, 128), jnp.float32),
    )(x, h1, h2, bt, w1, b1, w2, b2)


def kernel(x, edge_index, batch, W1, a1_src, a1_dst, b1, W2, a2_src, a2_dst, b2,
           mlp_w1, mlp_b1, mlp_w2, mlp_b2):
    src = edge_index[0]
    dst = edge_index[1]

    eye = jnp.eye(H, dtype=jnp.float32)
    As1 = jnp.einsum('hc,hk->hck', a1_src, eye).reshape(D, H)
    Ad1 = jnp.einsum('hc,hk->hck', a1_dst, eye).reshape(D, H)
    As2 = jnp.einsum('hc,hk->hck', a2_src, eye).reshape(D, H)
    Ad2 = jnp.einsum('hc,hk->hck', a2_dst, eye).reshape(D, H)
    sel = jnp.repeat(jnp.eye(H, dtype=jnp.float32), C, axis=1)

    z128 = jnp.zeros((N, D), jnp.float32)
    z4 = jnp.zeros((DENW,), jnp.float32)
    z1 = jnp.zeros((DEGW,), jnp.float32)
    b1_2d = b1.reshape(1, D)
    b2_2d = b2.reshape(1, D)
    bt = batch.reshape(1, N)

    h1p, als1, ald1 = _tc_proj(x, W1, As1, Ad1)
    denp1, degp1, ee1 = _sc_attn_pass(src, dst, als1.reshape(-1), ald1.reshape(-1), z4, z1)
    denp1 = denp1[:, 0, :N * H].reshape(NC, N, H)
    degp = degp1[:, 0, :N].reshape(NC, N, 1)
    accp1 = _sc_msg_pass(src, dst, h1p, ee1, z128)
    h1, g = _tc_gatfin(accp1, denp1, degp, b1_2d, sel)
    aggp = _sc_lp_pass(src, dst, g, z128)
    _, g = _tc_lpfin(aggp, h1, degp)
    aggp = _sc_lp_pass(src, dst, g, z128)
    h1f, _ = _tc_lpfin(aggp, h1, degp)

    h2p, als2, ald2 = _tc_proj(h1f, W2, As2, Ad2)
    denp2, _, ee2 = _sc_attn_pass(src, dst, als2.reshape(-1), ald2.reshape(-1), z4, z1)
    denp2 = denp2[:, 0, :N * H].reshape(NC, N, H)
    accp2 = _sc_msg_pass(src, dst, h2p, ee2, z128)
    h2, g = _tc_gatfin(accp2, denp2, degp, b2_2d, sel)
    aggp = _sc_lp_pass(src, dst, g, z128)
    _, g = _tc_lpfin(aggp, h2, degp)
    aggp = _sc_lp_pass(src, dst, g, z128)
    h2f, _ = _tc_lpfin(aggp, h2, degp)

    return _tc_final(x, h1f, h2f, bt, mlp_w1, mlp_b1.reshape(1, 256),
                     mlp_w2, mlp_b2.reshape(1, 128))


# speedup vs baseline: 47.8854x; 1.4848x over previous
import functools

import jax
import jax.numpy as jnp
from jax import lax
from jax.experimental import pallas as pl
from jax.experimental.pallas import tpu as pltpu
from jax.experimental.pallas import tpu_sc as plsc

N = 10000
E = 320000
D = 128
H = 4
C = 32
B = 64

NC = 2
NS = 16
L = 16

KCH = 80
EPT = E // (NC * NS)
NCHUNK = EPT // KCH
ROWS_PT = 624
TAIL0 = NS * ROWS_PT
TAILN = N - TAIL0

_MESH = plsc.VectorSubcoreMesh(core_axis_name="c", subcore_axis_name="s")


DENW = 40960
DEN_PT = DENW // NS
DEGW = 10240
DEG_PT = DEGW // NS


def _sc_attn_body(src_hbm, dst_hbm, als_hbm, ald_hbm, z4_hbm, z1_hbm,
                  den_out, deg_out, ee_out,
                  als_v, ald_v, srcb, dstb, ee_st, idx0, idx1, idx2, idx3,
                  ones_v, den_sh, deg_sh, sem):
    cid = lax.axis_index("c")
    sid = lax.axis_index("s")
    wid = sid * NC + cid

    pltpu.sync_copy(als_hbm, als_v)
    pltpu.sync_copy(ald_hbm, ald_v)

    d0 = pl.multiple_of(sid * DEN_PT, 128)
    pltpu.sync_copy(z4_hbm.at[pl.ds(d0, DEN_PT)], den_sh.at[pl.ds(d0, DEN_PT)])
    r0 = pl.multiple_of(sid * DEG_PT, 128)
    pltpu.sync_copy(z1_hbm.at[pl.ds(r0, DEG_PT)], deg_sh.at[pl.ds(r0, DEG_PT)])

    ones16 = jnp.full((L,), 1.0, jnp.float32)
    for g in range(KCH // L):
        ones_v[pl.ds(g * L, L)] = ones16

    plsc.subcore_barrier()

    ebase = wid * EPT
    idxs = (idx0, idx1, idx2, idx3)

    def chunk_body(c, carry):
        base = pl.multiple_of(ebase + c * KCH, 16)
        pltpu.sync_copy(src_hbm.at[pl.ds(base, KCH)], srcb)
        pltpu.sync_copy(dst_hbm.at[pl.ds(base, KCH)], dstb)

        for g in range(KCH // L):
            s16 = srcb[pl.ds(g * L, L)]
            d16 = dstb[pl.ds(g * L, L)]
            s4 = s16 * H
            d4 = d16 * H
            for hh in range(H):
                a_s = plsc.load_gather(als_v, [s4 + hh])
                a_d = plsc.load_gather(ald_v, [d4 + hh])
                z = a_s + a_d
                ee = jnp.exp(jnp.maximum(z, 0.2 * z))
                ee_st[pl.ds(hh * KCH + g * L, L)] = ee
                idxs[hh][pl.ds(g * L, L)] = d4 + hh

        for hh in range(H):
            pltpu.sync_copy(ee_st.at[pl.ds(hh * KCH, KCH)],
                            den_sh.at[idxs[hh]], add=True)
        pltpu.sync_copy(ones_v, deg_sh.at[dstb], add=True)
        pltpu.sync_copy(ee_st, ee_out.at[pl.ds(pl.multiple_of(base * H, 64), KCH * H)])
        return carry

    lax.fori_loop(0, NCHUNK, chunk_body, 0)

    plsc.subcore_barrier()

    pltpu.sync_copy(den_sh.at[pl.ds(d0, DEN_PT)], den_out.at[cid, 0, pl.ds(d0, DEN_PT)])
    pltpu.sync_copy(deg_sh.at[pl.ds(r0, DEG_PT)], deg_out.at[cid, 0, pl.ds(r0, DEG_PT)])


_sc_attn_pass = pl.kernel(
    _sc_attn_body,
    out_type=(
        jax.ShapeDtypeStruct((NC, 1, DENW), jnp.float32),
        jax.ShapeDtypeStruct((NC, 1, DEGW), jnp.float32),
        jax.ShapeDtypeStruct((E * H,), jnp.float32),
    ),
    mesh=_MESH,
    scratch_types=[
        pltpu.VMEM((N * H,), jnp.float32),
        pltpu.VMEM((N * H,), jnp.float32),
        pltpu.VMEM((KCH,), jnp.int32),
        pltpu.VMEM((KCH,), jnp.int32),
        pltpu.VMEM((KCH * H,), jnp.float32),
        pltpu.VMEM((KCH,), jnp.int32),
        pltpu.VMEM((KCH,), jnp.int32),
        pltpu.VMEM((KCH,), jnp.int32),
        pltpu.VMEM((KCH,), jnp.int32),
        pltpu.VMEM((KCH,), jnp.float32),
        pltpu.VMEM_SHARED((DENW,), jnp.float32),
        pltpu.VMEM_SHARED((DEGW,), jnp.float32),
        pltpu.SemaphoreType.DMA,
    ],
    compiler_params=pltpu.CompilerParams(needs_layout_passes=False),
)


KM = 40
NCH_M = EPT // KM


def _msg_scale(rows, ee_v):
    evs = [ee_v[pl.ds(v * L, L)] for v in range(KM * H // L)]
    for i in range(KM):
        for hh in range(H):
            j = hh * KM + i
            sv = jnp.full((L,), evs[j // L][j % L])
            for half in range(2):
                sl = pl.ds(hh * C + half * L, L)
                rows[i, sl] = rows[i, sl] * sv


def _sc_msg_body(src_hbm, dst_hbm, h_hbm, ee_hbm, z128_hbm,
                 acc_out,
                 srcb0, dstb0, eeb0, srcb1, dstb1, eeb1, rows0, rows1,
                 acc_sh, gsem0, gsem1, isem0, isem1):
    cid = lax.axis_index("c")
    sid = lax.axis_index("s")
    wid = sid * NC + cid

    r0 = sid * ROWS_PT
    pltpu.sync_copy(z128_hbm.at[pl.ds(r0, ROWS_PT)], acc_sh.at[pl.ds(r0, ROWS_PT)])

    @pl.when(sid == NS - 1)
    def _():
        pltpu.sync_copy(z128_hbm.at[pl.ds(TAIL0, TAILN)], acc_sh.at[pl.ds(TAIL0, TAILN)])

    plsc.subcore_barrier()

    ebase = wid * EPT

    def issue_idx(c, sb, db, eb, sem):
        base = pl.multiple_of(ebase + c * KM, 8)
        pltpu.async_copy(src_hbm.at[pl.ds(base, KM)], sb, sem)
        pltpu.async_copy(dst_hbm.at[pl.ds(base, KM)], db, sem)
        cb = c // 2
        half = c - cb * 2
        ebb = ebase * H + cb * (KCH * H) + half * KM
        for hh in range(H):
            pltpu.async_copy(
                ee_hbm.at[pl.ds(pl.multiple_of(ebb + hh * KCH, 8), KM)],
                eb.at[pl.ds(hh * KM, KM)], sem)

    def wait_idx(sb, db, eb, sem):
        pltpu.make_async_copy(src_hbm.at[pl.ds(0, KM)], sb, sem).wait()
        pltpu.make_async_copy(dst_hbm.at[pl.ds(0, KM)], db, sem).wait()
        for hh in range(H):
            pltpu.make_async_copy(ee_hbm.at[pl.ds(0, KM)],
                                  eb.at[pl.ds(hh * KM, KM)], sem).wait()

    issue_idx(0, srcb0, dstb0, eeb0, isem0)
    wait_idx(srcb0, dstb0, eeb0, isem0)
    pltpu.async_copy(h_hbm.at[srcb0], rows0, gsem0)
    issue_idx(1, srcb1, dstb1, eeb1, isem1)

    def pair_body(cc, carry):
        c0 = cc * 2
        pltpu.make_async_copy(h_hbm.at[srcb0], rows0, gsem0).wait()
        wait_idx(srcb1, dstb1, eeb1, isem1)
        pltpu.async_copy(h_hbm.at[srcb1], rows1, gsem1)
        _msg_scale(rows0, eeb0)
        pltpu.sync_copy(rows0, acc_sh.at[dstb0], add=True)
        issue_idx(jnp.minimum(c0 + 2, NCH_M - 1), srcb0, dstb0, eeb0, isem0)
        pltpu.make_async_copy(h_hbm.at[srcb1], rows1, gsem1).wait()
        _msg_scale(rows1, eeb1)
        pltpu.sync_copy(rows1, acc_sh.at[dstb1], add=True)
        wait_idx(srcb0, dstb0, eeb0, isem0)
        pltpu.async_copy(h_hbm.at[srcb0], rows0, gsem0)
        issue_idx(jnp.minimum(c0 + 3, NCH_M - 1), srcb1, dstb1, eeb1, isem1)
        return carry

    lax.fori_loop(0, NCH_M // 2, pair_body, 0)

    pltpu.make_async_copy(h_hbm.at[srcb0], rows0, gsem0).wait()
    wait_idx(srcb1, dstb1, eeb1, isem1)

    plsc.subcore_barrier()

    pltpu.sync_copy(acc_sh.at[pl.ds(r0, ROWS_PT)], acc_out.at[cid, pl.ds(r0, ROWS_PT)])

    @pl.when(sid == NS - 1)
    def _():
        pltpu.sync_copy(acc_sh.at[pl.ds(TAIL0, TAILN)], acc_out.at[cid, pl.ds(TAIL0, TAILN)])


_sc_msg_pass = pl.kernel(
    _sc_msg_body,
    out_type=jax.ShapeDtypeStruct((NC, N, D), jnp.float32),
    mesh=_MESH,
    scratch_types=[
        pltpu.VMEM((KM,), jnp.int32),
        pltpu.VMEM((KM,), jnp.int32),
        pltpu.VMEM((KM * H,), jnp.float32),
        pltpu.VMEM((KM,), jnp.int32),
        pltpu.VMEM((KM,), jnp.int32),
        pltpu.VMEM((KM * H,), jnp.float32),
        pltpu.VMEM((KM, D), jnp.float32),
        pltpu.VMEM((KM, D), jnp.float32),
        pltpu.VMEM_SHARED((N, D), jnp.float32),
        pltpu.SemaphoreType.DMA,
        pltpu.SemaphoreType.DMA,
        pltpu.SemaphoreType.DMA,
        pltpu.SemaphoreType.DMA,
    ],
    compiler_params=pltpu.CompilerParams(needs_layout_passes=False),
)


KL = 128
NCH_L = EPT // KL
LTAIL = EPT - NCH_L * KL


def _sc_lp_body(src_hbm, dst_hbm, g_hbm, z128_hbm, agg_out,
                srcb0, dstb0, srcb1, dstb1, rows0, rows1, srct, dstt, rowst,
                acc_sh, gsem0, gsem1, isem0, isem1, tsem):
    cid = lax.axis_index("c")
    sid = lax.axis_index("s")
    wid = sid * NC + cid

    r0 = sid * ROWS_PT
    pltpu.sync_copy(z128_hbm.at[pl.ds(r0, ROWS_PT)], acc_sh.at[pl.ds(r0, ROWS_PT)])

    @pl.when(sid == NS - 1)
    def _():
        pltpu.sync_copy(z128_hbm.at[pl.ds(TAIL0, TAILN)], acc_sh.at[pl.ds(TAIL0, TAILN)])

    plsc.subcore_barrier()

    ebase = wid * EPT

    def issue_idx(c, sb, db, sem):
        base = pl.multiple_of(ebase + c * KL, 8)
        pltpu.async_copy(src_hbm.at[pl.ds(base, KL)], sb, sem)
        pltpu.async_copy(dst_hbm.at[pl.ds(base, KL)], db, sem)

    def wait_idx(sb, db, sem):
        pltpu.make_async_copy(src_hbm.at[pl.ds(0, KL)], sb, sem).wait()
        pltpu.make_async_copy(dst_hbm.at[pl.ds(0, KL)], db, sem).wait()

    issue_idx(0, srcb0, dstb0, isem0)
    wait_idx(srcb0, dstb0, isem0)
    pltpu.async_copy(g_hbm.at[srcb0], rows0, gsem0)
    issue_idx(1, srcb1, dstb1, isem1)

    def pair_body(cc, carry):
        c0 = cc * 2
        pltpu.make_async_copy(g_hbm.at[srcb0], rows0, gsem0).wait()
        wait_idx(srcb1, dstb1, isem1)
        pltpu.async_copy(g_hbm.at[srcb1], rows1, gsem1)
        pltpu.sync_copy(rows0, acc_sh.at[dstb0], add=True)
        issue_idx(jnp.minimum(c0 + 2, NCH_L - 1), srcb0, dstb0, isem0)
        pltpu.make_async_copy(g_hbm.at[srcb1], rows1, gsem1).wait()
        pltpu.sync_copy(rows1, acc_sh.at[dstb1], add=True)
        wait_idx(srcb0, dstb0, isem0)
        pltpu.async_copy(g_hbm.at[srcb0], rows0, gsem0)
        issue_idx(jnp.minimum(c0 + 3, NCH_L - 1), srcb1, dstb1, isem1)
        return carry

    lax.fori_loop(0, NCH_L // 2, pair_body, 0)

    pltpu.make_async_copy(g_hbm.at[srcb0], rows0, gsem0).wait()
    wait_idx(srcb1, dstb1, isem1)

    tbase = pl.multiple_of(ebase + NCH_L * KL, 8)
    pltpu.sync_copy(src_hbm.at[pl.ds(tbase, LTAIL)], srct)
    pltpu.sync_copy(dst_hbm.at[pl.ds(tbase, LTAIL)], dstt)
    pltpu.async_copy(g_hbm.at[srct], rowst, tsem).wait()
    pltpu.sync_copy(rowst, acc_sh.at[dstt], add=True)

    plsc.subcore_barrier()

    pltpu.sync_copy(acc_sh.at[pl.ds(r0, ROWS_PT)], agg_out.at[cid, pl.ds(r0, ROWS_PT)])

    @pl.when(sid == NS - 1)
    def _():
        pltpu.sync_copy(acc_sh.at[pl.ds(TAIL0, TAILN)], agg_out.at[cid, pl.ds(TAIL0, TAILN)])


_sc_lp_pass = pl.kernel(
    _sc_lp_body,
    out_type=jax.ShapeDtypeStruct((NC, N, D), jnp.float32),
    mesh=_MESH,
    scratch_types=[
        pltpu.VMEM((KL,), jnp.int32),
        pltpu.VMEM((KL,), jnp.int32),
        pltpu.VMEM((KL,), jnp.int32),
        pltpu.VMEM((KL,), jnp.int32),
        pltpu.VMEM((KL, D), jnp.float32),
        pltpu.VMEM((KL, D), jnp.float32),
        pltpu.VMEM((LTAIL,), jnp.int32),
        pltpu.VMEM((LTAIL,), jnp.int32),
        pltpu.VMEM((LTAIL, D), jnp.float32),
        pltpu.VMEM_SHARED((N, D), jnp.float32),
        pltpu.SemaphoreType.DMA,
        pltpu.SemaphoreType.DMA,
        pltpu.SemaphoreType.DMA,
        pltpu.SemaphoreType.DMA,
        pltpu.SemaphoreType.DMA,
    ],
    compiler_params=pltpu.CompilerParams(needs_layout_passes=False),
)


def _tc_proj_body(x_ref, w_ref, as_ref, ad_ref, h_ref, als_ref, ald_ref):
    h = jnp.dot(x_ref[...], w_ref[...], preferred_element_type=jnp.float32)
    h_ref[...] = h
    als_ref[...] = jnp.dot(h, as_ref[...], preferred_element_type=jnp.float32)
    ald_ref[...] = jnp.dot(h, ad_ref[...], preferred_element_type=jnp.float32)


def _tc_proj(x, w, a_s, a_d):
    return pl.pallas_call(
        _tc_proj_body,
        out_shape=(
            jax.ShapeDtypeStruct((N, D), jnp.float32),
            jax.ShapeDtypeStruct((N, H), jnp.float32),
            jax.ShapeDtypeStruct((N, H), jnp.float32),
        ),
    )(x, w, a_s, a_d)


def _dis_from_deg(degp):
    deg = degp[0] + degp[1]
    return jnp.where(deg > 0, lax.rsqrt(jnp.maximum(deg, 1e-12)), 0.0)


def _tc_gatfin_body(accp_ref, denp_ref, degp_ref, b_ref, s_ref, h_ref, g_ref):
    acc = accp_ref[0] + accp_ref[1]
    den = denp_ref[0] + denp_ref[1]
    r = 1.0 / (den + 1e-16)
    r_rep = jnp.dot(r, s_ref[...], preferred_element_type=jnp.float32)
    h = jnp.maximum(acc * r_rep + b_ref[...], 0.0)
    h_ref[...] = h
    g_ref[...] = h * _dis_from_deg(degp_ref)


def _tc_gatfin(accp, denp, degp, b2d, sel):
    return pl.pallas_call(
        _tc_gatfin_body,
        out_shape=(
            jax.ShapeDtypeStruct((N, D), jnp.float32),
            jax.ShapeDtypeStruct((N, D), jnp.float32),
        ),
    )(accp, denp, degp, b2d, sel)


def _tc_lpfin_body(aggp_ref, hres_ref, degp_ref, out_ref, g_ref):
    dis = _dis_from_deg(degp_ref)
    agg = aggp_ref[0] + aggp_ref[1]
    out = jnp.clip(0.5 * dis * agg + 0.5 * hres_ref[...], 0.0, 1.0)
    out_ref[...] = out
    g_ref[...] = out * dis


def _tc_lpfin(aggp, hres, degp):
    return pl.pallas_call(
        _tc_lpfin_body,
        out_shape=(
            jax.ShapeDtypeStruct((N, D), jnp.float32),
            jax.ShapeDtypeStruct((N, D), jnp.float32),
        ),
    )(aggp, hres, degp)


def _tc_final_body(x_ref, h1_ref, h2_ref, bt_ref, w1_ref, b1_ref, w2_ref, b2_ref,
                   out_ref):
    combined = jnp.concatenate([x_ref[...], h1_ref[...], h2_ref[...]], axis=-1)
    bt = bt_ref[...]
    oh = (lax.broadcasted_iota(jnp.int32, (B, N), 0) == bt).astype(jnp.float32)
    pooled_sum = jnp.dot(oh, combined, preferred_element_type=jnp.float32)
    counts = jnp.sum(oh, axis=1, keepdims=True)
    pooled = pooled_sum / jnp.maximum(counts, 1.0)
    hmid = jnp.maximum(
        jnp.dot(pooled, w1_ref[...], preferred_element_type=jnp.float32) + b1_ref[...],
        0.0)
    out_ref[...] = jnp.dot(hmid, w2_ref[...], preferred_element_type=jnp.float32) + b2_ref[...]


def _tc_final(x, h1, h2, bt, w1, b1, w2, b2):
    return pl.pallas_call(
        _tc_final_body,
        out_shape=jax.ShapeDtypeStruct((B, 128), jnp.float32),
    )(x, h1, h2, bt, w1, b1, w2, b2)


def kernel(x, edge_index, batch, W1, a1_src, a1_dst, b1, W2, a2_src, a2_dst, b2,
           mlp_w1, mlp_b1, mlp_w2, mlp_b2):
    src = edge_index[0]
    dst = edge_index[1]

    eye = jnp.eye(H, dtype=jnp.float32)
    As1 = jnp.einsum('hc,hk->hck', a1_src, eye).reshape(D, H)
    Ad1 = jnp.einsum('hc,hk->hck', a1_dst, eye).reshape(D, H)
    As2 = jnp.einsum('hc,hk->hck', a2_src, eye).reshape(D, H)
    Ad2 = jnp.einsum('hc,hk->hck', a2_dst, eye).reshape(D, H)
    sel = jnp.repeat(jnp.eye(H, dtype=jnp.float32), C, axis=1)

    z128 = jnp.zeros((N, D), jnp.float32)
    z4 = jnp.zeros((DENW,), jnp.float32)
    z1 = jnp.zeros((DEGW,), jnp.float32)
    b1_2d = b1.reshape(1, D)
    b2_2d = b2.reshape(1, D)
    bt = batch.reshape(1, N)

    h1p, als1, ald1 = _tc_proj(x, W1, As1, Ad1)
    denp1, degp1, ee1 = _sc_attn_pass(src, dst, als1.reshape(-1), ald1.reshape(-1), z4, z1)
    denp1 = denp1[:, 0, :N * H].reshape(NC, N, H)
    degp = degp1[:, 0, :N].reshape(NC, N, 1)
    accp1 = _sc_msg_pass(src, dst, h1p, ee1, z128)
    h1, g = _tc_gatfin(accp1, denp1, degp, b1_2d, sel)
    aggp = _sc_lp_pass(src, dst, g, z128)
    _, g = _tc_lpfin(aggp, h1, degp)
    aggp = _sc_lp_pass(src, dst, g, z128)
    h1f, _ = _tc_lpfin(aggp, h1, degp)

    h2p, als2, ald2 = _tc_proj(h1f, W2, As2, Ad2)
    denp2, _, ee2 = _sc_attn_pass(src, dst, als2.reshape(-1), ald2.reshape(-1), z4, z1)
    denp2 = denp2[:, 0, :N * H].reshape(NC, N, H)
    accp2 = _sc_msg_pass(src, dst, h2p, ee2, z128)
    h2, g = _tc_gatfin(accp2, denp2, degp, b2_2d, sel)
    aggp = _sc_lp_pass(src, dst, g, z128)
    _, g = _tc_lpfin(aggp, h2, degp)
    aggp = _sc_lp_pass(src, dst, g, z128)
    h2f, _ = _tc_lpfin(aggp, h2, degp)

    return _tc_final(x, h1f, h2f, bt, mlp_w1, mlp_b1.reshape(1, 256),
                     mlp_w2, mlp_b2.reshape(1, 128))


# speedup vs baseline: 54.5440x; 1.1391x over previous
import functools

import jax
import jax.numpy as jnp
from jax import lax
from jax.experimental import pallas as pl
from jax.experimental.pallas import tpu as pltpu
from jax.experimental.pallas import tpu_sc as plsc

N = 10000
E = 320000
D = 128
H = 4
C = 32
B = 64

NC = 2
NS = 16
L = 16

KCH = 80
EPT = E // (NC * NS)
NCHUNK = EPT // KCH
ROWS_PT = 624
TAIL0 = NS * ROWS_PT
TAILN = N - TAIL0

_MESH = plsc.VectorSubcoreMesh(core_axis_name="c", subcore_axis_name="s")


DENW = 40960
DEN_PT = DENW // NS
DEGW = 10240
DEG_PT = DEGW // NS


def _sc_attn_body(src_hbm, dst_hbm, als_hbm, ald_hbm, z4_hbm, z1_hbm,
                  den_out, deg_out, ee_out,
                  als_v, ald_v, srcb0, dstb0, srcb1, dstb1, ee0, ee1,
                  idx0, idx1, idx2, idx3, ones_v, den_sh, deg_sh,
                  isem0, isem1, esem0, esem1):
    cid = lax.axis_index("c")
    sid = lax.axis_index("s")
    wid = sid * NC + cid

    pltpu.sync_copy(als_hbm, als_v)
    pltpu.sync_copy(ald_hbm, ald_v)

    d0 = pl.multiple_of(sid * DEN_PT, 128)
    pltpu.sync_copy(z4_hbm.at[pl.ds(d0, DEN_PT)], den_sh.at[pl.ds(d0, DEN_PT)])
    r0 = pl.multiple_of(sid * DEG_PT, 128)
    pltpu.sync_copy(z1_hbm.at[pl.ds(r0, DEG_PT)], deg_sh.at[pl.ds(r0, DEG_PT)])

    ones16 = jnp.full((L,), 1.0, jnp.float32)
    for g in range(KCH // L):
        ones_v[pl.ds(g * L, L)] = ones16

    plsc.subcore_barrier()

    ebase = wid * EPT
    idxs = (idx0, idx1, idx2, idx3)

    def issue_idx(c, sb, db, sem):
        base = pl.multiple_of(ebase + c * KCH, 16)
        pltpu.async_copy(src_hbm.at[pl.ds(base, KCH)], sb, sem)
        pltpu.async_copy(dst_hbm.at[pl.ds(base, KCH)], db, sem)

    def wait_idx(sb, db, sem):
        pltpu.make_async_copy(src_hbm.at[pl.ds(0, KCH)], sb, sem).wait()
        pltpu.make_async_copy(dst_hbm.at[pl.ds(0, KCH)], db, sem).wait()

    def compute(c, sb, db, ee_st):
        for g in range(KCH // L):
            s16 = sb[pl.ds(g * L, L)]
            d16 = db[pl.ds(g * L, L)]
            s4 = s16 * H
            d4 = d16 * H
            for hh in range(H):
                a_s = plsc.load_gather(als_v, [s4 + hh])
                a_d = plsc.load_gather(ald_v, [d4 + hh])
                z = a_s + a_d
                ee = jnp.exp(jnp.maximum(z, 0.2 * z))
                ee_st[pl.ds(hh * KCH + g * L, L)] = ee
                idxs[hh][pl.ds(g * L, L)] = d4 + hh
        for hh in range(H):
            pltpu.sync_copy(ee_st.at[pl.ds(hh * KCH, KCH)],
                            den_sh.at[idxs[hh]], add=True)
        pltpu.sync_copy(ones_v, deg_sh.at[db], add=True)

    def write_ee(c, ee_st, sem):
        base = pl.multiple_of((ebase + c * KCH) * H, 64)
        pltpu.async_copy(ee_st, ee_out.at[pl.ds(base, KCH * H)], sem)

    def drain_ee(ee_st, sem):
        pltpu.make_async_copy(ee_st, ee_out.at[pl.ds(0, KCH * H)], sem).wait()

    issue_idx(0, srcb0, dstb0, isem0)
    issue_idx(1, srcb1, dstb1, isem1)
    wait_idx(srcb0, dstb0, isem0)
    compute(0, srcb0, dstb0, ee0)
    write_ee(0, ee0, esem0)
    issue_idx(2, srcb0, dstb0, isem0)
    wait_idx(srcb1, dstb1, isem1)
    compute(1, srcb1, dstb1, ee1)
    write_ee(1, ee1, esem1)
    issue_idx(3, srcb1, dstb1, isem1)

    def pair_body(cc, carry):
        c0 = cc * 2
        wait_idx(srcb0, dstb0, isem0)
        drain_ee(ee0, esem0)
        compute(c0, srcb0, dstb0, ee0)
        write_ee(c0, ee0, esem0)
        issue_idx(jnp.minimum(c0 + 2, NCHUNK - 1), srcb0, dstb0, isem0)
        wait_idx(srcb1, dstb1, isem1)
        drain_ee(ee1, esem1)
        compute(c0 + 1, srcb1, dstb1, ee1)
        write_ee(c0 + 1, ee1, esem1)
        issue_idx(jnp.minimum(c0 + 3, NCHUNK - 1), srcb1, dstb1, isem1)
        return carry

    lax.fori_loop(1, (NCHUNK - 1) // 2, pair_body, 0)

    wait_idx(srcb0, dstb0, isem0)
    drain_ee(ee0, esem0)
    compute(NCHUNK - 1, srcb0, dstb0, ee0)
    write_ee(NCHUNK - 1, ee0, esem0)
    drain_ee(ee0, esem0)
    drain_ee(ee1, esem1)
    wait_idx(srcb1, dstb1, isem1)

    plsc.subcore_barrier()

    pltpu.sync_copy(den_sh.at[pl.ds(d0, DEN_PT)], den_out.at[cid, 0, pl.ds(d0, DEN_PT)])
    pltpu.sync_copy(deg_sh.at[pl.ds(r0, DEG_PT)], deg_out.at[cid, 0, pl.ds(r0, DEG_PT)])


_sc_attn_pass = pl.kernel(
    _sc_attn_body,
    out_type=(
        jax.ShapeDtypeStruct((NC, 1, DENW), jnp.float32),
        jax.ShapeDtypeStruct((NC, 1, DEGW), jnp.float32),
        jax.ShapeDtypeStruct((E * H,), jnp.float32),
    ),
    mesh=_MESH,
    scratch_types=[
        pltpu.VMEM((N * H,), jnp.float32),
        pltpu.VMEM((N * H,), jnp.float32),
        pltpu.VMEM((KCH,), jnp.int32),
        pltpu.VMEM((KCH,), jnp.int32),
        pltpu.VMEM((KCH,), jnp.int32),
        pltpu.VMEM((KCH,), jnp.int32),
        pltpu.VMEM((KCH * H,), jnp.float32),
        pltpu.VMEM((KCH * H,), jnp.float32),
        pltpu.VMEM((KCH,), jnp.int32),
        pltpu.VMEM((KCH,), jnp.int32),
        pltpu.VMEM((KCH,), jnp.int32),
        pltpu.VMEM((KCH,), jnp.int32),
        pltpu.VMEM((KCH,), jnp.float32),
        pltpu.VMEM_SHARED((DENW,), jnp.float32),
        pltpu.VMEM_SHARED((DEGW,), jnp.float32),
        pltpu.SemaphoreType.DMA,
        pltpu.SemaphoreType.DMA,
        pltpu.SemaphoreType.DMA,
        pltpu.SemaphoreType.DMA,
    ],
    compiler_params=pltpu.CompilerParams(needs_layout_passes=False),
)


KM = 40
NCH_M = EPT // KM


def _msg_scale(rows, ee_v):
    evs = [ee_v[pl.ds(v * L, L)] for v in range(KM * H // L)]
    for i in range(KM):
        for hh in range(H):
            j = hh * KM + i
            sv = jnp.full((L,), evs[j // L][j % L])
            for half in range(2):
                sl = pl.ds(hh * C + half * L, L)
                rows[i, sl] = rows[i, sl] * sv


def _sc_msg_body(src_hbm, dst_hbm, h_hbm, ee_hbm, z128_hbm,
                 acc_out,
                 srcb0, dstb0, eeb0, srcb1, dstb1, eeb1, rows0, rows1,
                 acc_sh, gsem0, gsem1, isem0, isem1):
    cid = lax.axis_index("c")
    sid = lax.axis_index("s")
    wid = sid * NC + cid

    r0 = sid * ROWS_PT
    pltpu.sync_copy(z128_hbm.at[pl.ds(r0, ROWS_PT)], acc_sh.at[pl.ds(r0, ROWS_PT)])

    @pl.when(sid == NS - 1)
    def _():
        pltpu.sync_copy(z128_hbm.at[pl.ds(TAIL0, TAILN)], acc_sh.at[pl.ds(TAIL0, TAILN)])

    plsc.subcore_barrier()

    ebase = wid * EPT

    def issue_idx(c, sb, db, eb, sem):
        base = pl.multiple_of(ebase + c * KM, 8)
        pltpu.async_copy(src_hbm.at[pl.ds(base, KM)], sb, sem)
        pltpu.async_copy(dst_hbm.at[pl.ds(base, KM)], db, sem)
        cb = c // 2
        half = c - cb * 2
        ebb = ebase * H + cb * (KCH * H) + half * KM
        for hh in range(H):
            pltpu.async_copy(
                ee_hbm.at[pl.ds(pl.multiple_of(ebb + hh * KCH, 8), KM)],
                eb.at[pl.ds(hh * KM, KM)], sem)

    def wait_idx(sb, db, eb, sem):
        pltpu.make_async_copy(src_hbm.at[pl.ds(0, KM)], sb, sem).wait()
        pltpu.make_async_copy(dst_hbm.at[pl.ds(0, KM)], db, sem).wait()
        for hh in range(H):
            pltpu.make_async_copy(ee_hbm.at[pl.ds(0, KM)],
                                  eb.at[pl.ds(hh * KM, KM)], sem).wait()

    issue_idx(0, srcb0, dstb0, eeb0, isem0)
    wait_idx(srcb0, dstb0, eeb0, isem0)
    pltpu.async_copy(h_hbm.at[srcb0], rows0, gsem0)
    issue_idx(1, srcb1, dstb1, eeb1, isem1)

    def pair_body(cc, carry):
        c0 = cc * 2
        pltpu.make_async_copy(h_hbm.at[srcb0], rows0, gsem0).wait()
        wait_idx(srcb1, dstb1, eeb1, isem1)
        pltpu.async_copy(h_hbm.at[srcb1], rows1, gsem1)
        _msg_scale(rows0, eeb0)
        pltpu.sync_copy(rows0, acc_sh.at[dstb0], add=True)
        issue_idx(jnp.minimum(c0 + 2, NCH_M - 1), srcb0, dstb0, eeb0, isem0)
        pltpu.make_async_copy(h_hbm.at[srcb1], rows1, gsem1).wait()
        _msg_scale(rows1, eeb1)
        pltpu.sync_copy(rows1, acc_sh.at[dstb1], add=True)
        wait_idx(srcb0, dstb0, eeb0, isem0)
        pltpu.async_copy(h_hbm.at[srcb0], rows0, gsem0)
        issue_idx(jnp.minimum(c0 + 3, NCH_M - 1), srcb1, dstb1, eeb1, isem1)
        return carry

    lax.fori_loop(0, NCH_M // 2, pair_body, 0)

    pltpu.make_async_copy(h_hbm.at[srcb0], rows0, gsem0).wait()
    wait_idx(srcb1, dstb1, eeb1, isem1)

    plsc.subcore_barrier()

    pltpu.sync_copy(acc_sh.at[pl.ds(r0, ROWS_PT)], acc_out.at[cid, pl.ds(r0, ROWS_PT)])

    @pl.when(sid == NS - 1)
    def _():
        pltpu.sync_copy(acc_sh.at[pl.ds(TAIL0, TAILN)], acc_out.at[cid, pl.ds(TAIL0, TAILN)])


_sc_msg_pass = pl.kernel(
    _sc_msg_body,
    out_type=jax.ShapeDtypeStruct((NC, N, D), jnp.float32),
    mesh=_MESH,
    scratch_types=[
        pltpu.VMEM((KM,), jnp.int32),
        pltpu.VMEM((KM,), jnp.int32),
        pltpu.VMEM((KM * H,), jnp.float32),
        pltpu.VMEM((KM,), jnp.int32),
        pltpu.VMEM((KM,), jnp.int32),
        pltpu.VMEM((KM * H,), jnp.float32),
        pltpu.VMEM((KM, D), jnp.float32),
        pltpu.VMEM((KM, D), jnp.float32),
        pltpu.VMEM_SHARED((N, D), jnp.float32),
        pltpu.SemaphoreType.DMA,
        pltpu.SemaphoreType.DMA,
        pltpu.SemaphoreType.DMA,
        pltpu.SemaphoreType.DMA,
    ],
    compiler_params=pltpu.CompilerParams(needs_layout_passes=False),
)


KL = 128
NCH_L = EPT // KL
LTAIL = EPT - NCH_L * KL


def _sc_lp_body(src_hbm, dst_hbm, g_hbm, z128_hbm, agg_out,
                srcb0, dstb0, srcb1, dstb1, rows0, rows1, srct, dstt, rowst,
                acc_sh, gsem0, gsem1, isem0, isem1, tsem):
    cid = lax.axis_index("c")
    sid = lax.axis_index("s")
    wid = sid * NC + cid

    r0 = sid * ROWS_PT
    pltpu.sync_copy(z128_hbm.at[pl.ds(r0, ROWS_PT)], acc_sh.at[pl.ds(r0, ROWS_PT)])

    @pl.when(sid == NS - 1)
    def _():
        pltpu.sync_copy(z128_hbm.at[pl.ds(TAIL0, TAILN)], acc_sh.at[pl.ds(TAIL0, TAILN)])

    plsc.subcore_barrier()

    ebase = wid * EPT

    def issue_idx(c, sb, db, sem):
        base = pl.multiple_of(ebase + c * KL, 8)
        pltpu.async_copy(src_hbm.at[pl.ds(base, KL)], sb, sem)
        pltpu.async_copy(dst_hbm.at[pl.ds(base, KL)], db, sem)

    def wait_idx(sb, db, sem):
        pltpu.make_async_copy(src_hbm.at[pl.ds(0, KL)], sb, sem).wait()
        pltpu.make_async_copy(dst_hbm.at[pl.ds(0, KL)], db, sem).wait()

    issue_idx(0, srcb0, dstb0, isem0)
    wait_idx(srcb0, dstb0, isem0)
    pltpu.async_copy(g_hbm.at[srcb0], rows0, gsem0)
    issue_idx(1, srcb1, dstb1, isem1)

    def pair_body(cc, carry):
        c0 = cc * 2
        pltpu.make_async_copy(g_hbm.at[srcb0], rows0, gsem0).wait()
        wait_idx(srcb1, dstb1, isem1)
        pltpu.async_copy(g_hbm.at[srcb1], rows1, gsem1)
        pltpu.sync_copy(rows0, acc_sh.at[dstb0], add=True)
        issue_idx(jnp.minimum(c0 + 2, NCH_L - 1), srcb0, dstb0, isem0)
        pltpu.make_async_copy(g_hbm.at[srcb1], rows1, gsem1).wait()
        pltpu.sync_copy(rows1, acc_sh.at[dstb1], add=True)
        wait_idx(srcb0, dstb0, isem0)
        pltpu.async_copy(g_hbm.at[srcb0], rows0, gsem0)
        issue_idx(jnp.minimum(c0 + 3, NCH_L - 1), srcb1, dstb1, isem1)
        return carry

    lax.fori_loop(0, NCH_L // 2, pair_body, 0)

    pltpu.make_async_copy(g_hbm.at[srcb0], rows0, gsem0).wait()
    wait_idx(srcb1, dstb1, isem1)

    tbase = pl.multiple_of(ebase + NCH_L * KL, 8)
    pltpu.sync_copy(src_hbm.at[pl.ds(tbase, LTAIL)], srct)
    pltpu.sync_copy(dst_hbm.at[pl.ds(tbase, LTAIL)], dstt)
    pltpu.async_copy(g_hbm.at[srct], rowst, tsem).wait()
    pltpu.sync_copy(rowst, acc_sh.at[dstt], add=True)

    plsc.subcore_barrier()

    pltpu.sync_copy(acc_sh.at[pl.ds(r0, ROWS_PT)], agg_out.at[cid, pl.ds(r0, ROWS_PT)])

    @pl.when(sid == NS - 1)
    def _():
        pltpu.sync_copy(acc_sh.at[pl.ds(TAIL0, TAILN)], agg_out.at[cid, pl.ds(TAIL0, TAILN)])


_sc_lp_pass = pl.kernel(
    _sc_lp_body,
    out_type=jax.ShapeDtypeStruct((NC, N, D), jnp.float32),
    mesh=_MESH,
    scratch_types=[
        pltpu.VMEM((KL,), jnp.int32),
        pltpu.VMEM((KL,), jnp.int32),
        pltpu.VMEM((KL,), jnp.int32),
        pltpu.VMEM((KL,), jnp.int32),
        pltpu.VMEM((KL, D), jnp.float32),
        pltpu.VMEM((KL, D), jnp.float32),
        pltpu.VMEM((LTAIL,), jnp.int32),
        pltpu.VMEM((LTAIL,), jnp.int32),
        pltpu.VMEM((LTAIL, D), jnp.float32),
        pltpu.VMEM_SHARED((N, D), jnp.float32),
        pltpu.SemaphoreType.DMA,
        pltpu.SemaphoreType.DMA,
        pltpu.SemaphoreType.DMA,
        pltpu.SemaphoreType.DMA,
        pltpu.SemaphoreType.DMA,
    ],
    compiler_params=pltpu.CompilerParams(needs_layout_passes=False),
)


def _tc_proj_body(x_ref, w_ref, as_ref, ad_ref, h_ref, als_ref, ald_ref):
    h = jnp.dot(x_ref[...], w_ref[...], preferred_element_type=jnp.float32)
    h_ref[...] = h
    als_ref[...] = jnp.dot(h, as_ref[...], preferred_element_type=jnp.float32)
    ald_ref[...] = jnp.dot(h, ad_ref[...], preferred_element_type=jnp.float32)


def _tc_proj(x, w, a_s, a_d):
    return pl.pallas_call(
        _tc_proj_body,
        out_shape=(
            jax.ShapeDtypeStruct((N, D), jnp.float32),
            jax.ShapeDtypeStruct((N, H), jnp.float32),
            jax.ShapeDtypeStruct((N, H), jnp.float32),
        ),
    )(x, w, a_s, a_d)


def _dis_from_deg(degp):
    deg = degp[0] + degp[1]
    return jnp.where(deg > 0, lax.rsqrt(jnp.maximum(deg, 1e-12)), 0.0)


def _tc_gatfin_body(accp_ref, denp_ref, degp_ref, b_ref, s_ref, h_ref, g_ref):
    acc = accp_ref[0] + accp_ref[1]
    den = denp_ref[0] + denp_ref[1]
    r = 1.0 / (den + 1e-16)
    r_rep = jnp.dot(r, s_ref[...], preferred_element_type=jnp.float32)
    h = jnp.maximum(acc * r_rep + b_ref[...], 0.0)
    h_ref[...] = h
    g_ref[...] = h * _dis_from_deg(degp_ref)


def _tc_gatfin(accp, denp, degp, b2d, sel):
    return pl.pallas_call(
        _tc_gatfin_body,
        out_shape=(
            jax.ShapeDtypeStruct((N, D), jnp.float32),
            jax.ShapeDtypeStruct((N, D), jnp.float32),
        ),
    )(accp, denp, degp, b2d, sel)


def _tc_lpfin_body(aggp_ref, hres_ref, degp_ref, out_ref, g_ref):
    dis = _dis_from_deg(degp_ref)
    agg = aggp_ref[0] + aggp_ref[1]
    out = jnp.clip(0.5 * dis * agg + 0.5 * hres_ref[...], 0.0, 1.0)
    out_ref[...] = out
    g_ref[...] = out * dis


def _tc_lpfin(aggp, hres, degp):
    return pl.pallas_call(
        _tc_lpfin_body,
        out_shape=(
            jax.ShapeDtypeStruct((N, D), jnp.float32),
            jax.ShapeDtypeStruct((N, D), jnp.float32),
        ),
    )(aggp, hres, degp)


def _tc_final_body(x_ref, h1_ref, h2_ref, bt_ref, w1_ref, b1_ref, w2_ref, b2_ref,
                   out_ref):
    combined = jnp.concatenate([x_ref[...], h1_ref[...], h2_ref[...]], axis=-1)
    bt = bt_ref[...]
    oh = (lax.broadcasted_iota(jnp.int32, (B, N), 0) == bt).astype(jnp.float32)
    pooled_sum = jnp.dot(oh, combined, preferred_element_type=jnp.float32)
    counts = jnp.sum(oh, axis=1, keepdims=True)
    pooled = pooled_sum / jnp.maximum(counts, 1.0)
    hmid = jnp.maximum(
        jnp.dot(pooled, w1_ref[...], preferred_element_type=jnp.float32) + b1_ref[...],
        0.0)
    out_ref[...] = jnp.dot(hmid, w2_ref[...], preferred_element_type=jnp.float32) + b2_ref[...]


def _tc_final(x, h1, h2, bt, w1, b1, w2, b2):
    return pl.pallas_call(
        _tc_final_body,
        out_shape=jax.ShapeDtypeStruct((B, 128), jnp.float32),
    )(x, h1, h2, bt, w1, b1, w2, b2)


def kernel(x, edge_index, batch, W1, a1_src, a1_dst, b1, W2, a2_src, a2_dst, b2,
           mlp_w1, mlp_b1, mlp_w2, mlp_b2):
    src = edge_index[0]
    dst = edge_index[1]

    eye = jnp.eye(H, dtype=jnp.float32)
    As1 = jnp.einsum('hc,hk->hck', a1_src, eye).reshape(D, H)
    Ad1 = jnp.einsum('hc,hk->hck', a1_dst, eye).reshape(D, H)
    As2 = jnp.einsum('hc,hk->hck', a2_src, eye).reshape(D, H)
    Ad2 = jnp.einsum('hc,hk->hck', a2_dst, eye).reshape(D, H)
    sel = jnp.repeat(jnp.eye(H, dtype=jnp.float32), C, axis=1)

    z128 = jnp.zeros((N, D), jnp.float32)
    z4 = jnp.zeros((DENW,), jnp.float32)
    z1 = jnp.zeros((DEGW,), jnp.float32)
    b1_2d = b1.reshape(1, D)
    b2_2d = b2.reshape(1, D)
    bt = batch.reshape(1, N)

    h1p, als1, ald1 = _tc_proj(x, W1, As1, Ad1)
    denp1, degp1, ee1 = _sc_attn_pass(src, dst, als1.reshape(-1), ald1.reshape(-1), z4, z1)
    denp1 = denp1[:, 0, :N * H].reshape(NC, N, H)
    degp = degp1[:, 0, :N].reshape(NC, N, 1)
    accp1 = _sc_msg_pass(src, dst, h1p, ee1, z128)
    h1, g = _tc_gatfin(accp1, denp1, degp, b1_2d, sel)
    aggp = _sc_lp_pass(src, dst, g, z128)
    _, g = _tc_lpfin(aggp, h1, degp)
    aggp = _sc_lp_pass(src, dst, g, z128)
    h1f, _ = _tc_lpfin(aggp, h1, degp)

    h2p, als2, ald2 = _tc_proj(h1f, W2, As2, Ad2)
    denp2, _, ee2 = _sc_attn_pass(src, dst, als2.reshape(-1), ald2.reshape(-1), z4, z1)
    denp2 = denp2[:, 0, :N * H].reshape(NC, N, H)
    accp2 = _sc_msg_pass(src, dst, h2p, ee2, z128)
    h2, g = _tc_gatfin(accp2, denp2, degp, b2_2d, sel)
    aggp = _sc_lp_pass(src, dst, g, z128)
    _, g = _tc_lpfin(aggp, h2, degp)
    aggp = _sc_lp_pass(src, dst, g, z128)
    h2f, _ = _tc_lpfin(aggp, h2, degp)

    return _tc_final(x, h1f, h2f, bt, mlp_w1, mlp_b1.reshape(1, 256),
                     mlp_w2, mlp_b2.reshape(1, 128))


# speedup vs baseline: 54.5645x; 1.0004x over previous
import jax
import jax.numpy as jnp
from jax import lax
from jax.experimental import pallas as pl
from jax.experimental.pallas import tpu as pltpu
from jax.experimental.pallas import tpu_sc as plsc

N = 10000
E = 320000
D = 128
H = 4
C = 32
B = 64

NC = 2
NS = 16
L = 16

KCH = 80
EPT = E // (NC * NS)
NCHUNK = EPT // KCH
ROWS_PT = 624
TAIL0 = NS * ROWS_PT
TAILN = N - TAIL0

_MESH = plsc.VectorSubcoreMesh(core_axis_name="c", subcore_axis_name="s")


DENW = 40960
DEN_PT = DENW // NS
DEGW = 10240
DEG_PT = DEGW // NS


def _sc_attn_body(src_hbm, dst_hbm, als_hbm, ald_hbm, z4_hbm, z1_hbm,
                  den_out, deg_out, ee_out,
                  als_v, ald_v, srcb0, dstb0, srcb1, dstb1, ee0, ee1,
                  idx0, idx1, idx2, idx3, ones_v, den_sh, deg_sh,
                  isem0, isem1, esem0, esem1):
    cid = lax.axis_index("c")
    sid = lax.axis_index("s")
    wid = sid * NC + cid

    pltpu.sync_copy(als_hbm, als_v)
    pltpu.sync_copy(ald_hbm, ald_v)

    d0 = pl.multiple_of(sid * DEN_PT, 128)
    pltpu.sync_copy(z4_hbm.at[pl.ds(d0, DEN_PT)], den_sh.at[pl.ds(d0, DEN_PT)])
    r0 = pl.multiple_of(sid * DEG_PT, 128)
    pltpu.sync_copy(z1_hbm.at[pl.ds(r0, DEG_PT)], deg_sh.at[pl.ds(r0, DEG_PT)])

    ones16 = jnp.full((L,), 1.0, jnp.float32)
    for g in range(KCH // L):
        ones_v[pl.ds(g * L, L)] = ones16

    plsc.subcore_barrier()

    ebase = wid * EPT
    idxs = (idx0, idx1, idx2, idx3)

    def issue_idx(c, sb, db, sem):
        base = pl.multiple_of(ebase + c * KCH, 16)
        pltpu.async_copy(src_hbm.at[pl.ds(base, KCH)], sb, sem)
        pltpu.async_copy(dst_hbm.at[pl.ds(base, KCH)], db, sem)

    def wait_idx(sb, db, sem):
        pltpu.make_async_copy(src_hbm.at[pl.ds(0, KCH)], sb, sem).wait()
        pltpu.make_async_copy(dst_hbm.at[pl.ds(0, KCH)], db, sem).wait()

    def compute(c, sb, db, ee_st):
        for g in range(KCH // L):
            s16 = sb[pl.ds(g * L, L)]
            d16 = db[pl.ds(g * L, L)]
            s4 = s16 * H
            d4 = d16 * H
            for hh in range(H):
                a_s = plsc.load_gather(als_v, [s4 + hh])
                a_d = plsc.load_gather(ald_v, [d4 + hh])
                z = a_s + a_d
                ee = jnp.exp(jnp.maximum(z, 0.2 * z))
                ee_st[pl.ds(hh * KCH + g * L, L)] = ee
                idxs[hh][pl.ds(g * L, L)] = d4 + hh
        for hh in range(H):
            pltpu.sync_copy(ee_st.at[pl.ds(hh * KCH, KCH)],
                            den_sh.at[idxs[hh]], add=True)
        pltpu.sync_copy(ones_v, deg_sh.at[db], add=True)

    def write_ee(c, ee_st, sem):
        base = pl.multiple_of((ebase + c * KCH) * H, 64)
        pltpu.async_copy(ee_st, ee_out.at[pl.ds(base, KCH * H)], sem)

    def drain_ee(ee_st, sem):
        pltpu.make_async_copy(ee_st, ee_out.at[pl.ds(0, KCH * H)], sem).wait()

    issue_idx(0, srcb0, dstb0, isem0)
    issue_idx(1, srcb1, dstb1, isem1)
    wait_idx(srcb0, dstb0, isem0)
    compute(0, srcb0, dstb0, ee0)
    write_ee(0, ee0, esem0)
    issue_idx(2, srcb0, dstb0, isem0)
    wait_idx(srcb1, dstb1, isem1)
    compute(1, srcb1, dstb1, ee1)
    write_ee(1, ee1, esem1)
    issue_idx(3, srcb1, dstb1, isem1)

    def pair_body(cc, carry):
        c0 = cc * 2
        wait_idx(srcb0, dstb0, isem0)
        drain_ee(ee0, esem0)
        compute(c0, srcb0, dstb0, ee0)
        write_ee(c0, ee0, esem0)
        issue_idx(jnp.minimum(c0 + 2, NCHUNK - 1), srcb0, dstb0, isem0)
        wait_idx(srcb1, dstb1, isem1)
        drain_ee(ee1, esem1)
        compute(c0 + 1, srcb1, dstb1, ee1)
        write_ee(c0 + 1, ee1, esem1)
        issue_idx(jnp.minimum(c0 + 3, NCHUNK - 1), srcb1, dstb1, isem1)
        return carry

    lax.fori_loop(1, (NCHUNK - 1) // 2, pair_body, 0)

    wait_idx(srcb0, dstb0, isem0)
    drain_ee(ee0, esem0)
    compute(NCHUNK - 1, srcb0, dstb0, ee0)
    write_ee(NCHUNK - 1, ee0, esem0)
    drain_ee(ee0, esem0)
    drain_ee(ee1, esem1)
    wait_idx(srcb1, dstb1, isem1)

    plsc.subcore_barrier()

    pltpu.sync_copy(den_sh.at[pl.ds(d0, DEN_PT)], den_out.at[cid, 0, pl.ds(d0, DEN_PT)])
    pltpu.sync_copy(deg_sh.at[pl.ds(r0, DEG_PT)], deg_out.at[cid, 0, pl.ds(r0, DEG_PT)])


_sc_attn_pass = pl.kernel(
    _sc_attn_body,
    out_type=(
        jax.ShapeDtypeStruct((NC, 1, DENW), jnp.float32),
        jax.ShapeDtypeStruct((NC, 1, DEGW), jnp.float32),
        jax.ShapeDtypeStruct((E * H,), jnp.float32),
    ),
    mesh=_MESH,
    scratch_types=[
        pltpu.VMEM((N * H,), jnp.float32),
        pltpu.VMEM((N * H,), jnp.float32),
        pltpu.VMEM((KCH,), jnp.int32),
        pltpu.VMEM((KCH,), jnp.int32),
        pltpu.VMEM((KCH,), jnp.int32),
        pltpu.VMEM((KCH,), jnp.int32),
        pltpu.VMEM((KCH * H,), jnp.float32),
        pltpu.VMEM((KCH * H,), jnp.float32),
        pltpu.VMEM((KCH,), jnp.int32),
        pltpu.VMEM((KCH,), jnp.int32),
        pltpu.VMEM((KCH,), jnp.int32),
        pltpu.VMEM((KCH,), jnp.int32),
        pltpu.VMEM((KCH,), jnp.float32),
        pltpu.VMEM_SHARED((DENW,), jnp.float32),
        pltpu.VMEM_SHARED((DEGW,), jnp.float32),
        pltpu.SemaphoreType.DMA,
        pltpu.SemaphoreType.DMA,
        pltpu.SemaphoreType.DMA,
        pltpu.SemaphoreType.DMA,
    ],
    compiler_params=pltpu.CompilerParams(needs_layout_passes=False),
)


KM = 40
NCH_M = EPT // KM


def _msg_scale(rows, ee_v):
    evs = [ee_v[pl.ds(v * L, L)] for v in range(KM * H // L)]
    for i in range(KM):
        for hh in range(H):
            j = hh * KM + i
            sv = jnp.full((L,), evs[j // L][j % L])
            for half in range(2):
                sl = pl.ds(hh * C + half * L, L)
                rows[i, sl] = rows[i, sl] * sv


def _sc_msg_body(src_hbm, dst_hbm, h_hbm, ee_hbm, z128_hbm,
                 acc_out,
                 srcb0, dstb0, eeb0, srcb1, dstb1, eeb1, rows0, rows1,
                 acc_sh, gsem0, gsem1, isem0, isem1):
    cid = lax.axis_index("c")
    sid = lax.axis_index("s")
    wid = sid * NC + cid

    r0 = sid * ROWS_PT
    pltpu.sync_copy(z128_hbm.at[pl.ds(r0, ROWS_PT)], acc_sh.at[pl.ds(r0, ROWS_PT)])

    @pl.when(sid == NS - 1)
    def _():
        pltpu.sync_copy(z128_hbm.at[pl.ds(TAIL0, TAILN)], acc_sh.at[pl.ds(TAIL0, TAILN)])

    plsc.subcore_barrier()

    ebase = wid * EPT

    def issue_idx(c, sb, db, eb, sem):
        base = pl.multiple_of(ebase + c * KM, 8)
        pltpu.async_copy(src_hbm.at[pl.ds(base, KM)], sb, sem)
        pltpu.async_copy(dst_hbm.at[pl.ds(base, KM)], db, sem)
        cb = c // 2
        half = c - cb * 2
        ebb = ebase * H + cb * (KCH * H) + half * KM
        for hh in range(H):
            pltpu.async_copy(
                ee_hbm.at[pl.ds(pl.multiple_of(ebb + hh * KCH, 8), KM)],
                eb.at[pl.ds(hh * KM, KM)], sem)

    def wait_idx(sb, db, eb, sem):
        pltpu.make_async_copy(src_hbm.at[pl.ds(0, KM)], sb, sem).wait()
        pltpu.make_async_copy(dst_hbm.at[pl.ds(0, KM)], db, sem).wait()
        for hh in range(H):
            pltpu.make_async_copy(ee_hbm.at[pl.ds(0, KM)],
                                  eb.at[pl.ds(hh * KM, KM)], sem).wait()

    issue_idx(0, srcb0, dstb0, eeb0, isem0)
    wait_idx(srcb0, dstb0, eeb0, isem0)
    pltpu.async_copy(h_hbm.at[srcb0], rows0, gsem0)
    issue_idx(1, srcb1, dstb1, eeb1, isem1)

    def pair_body(cc, carry):
        c0 = cc * 2
        pltpu.make_async_copy(h_hbm.at[srcb0], rows0, gsem0).wait()
        wait_idx(srcb1, dstb1, eeb1, isem1)
        pltpu.async_copy(h_hbm.at[srcb1], rows1, gsem1)
        _msg_scale(rows0, eeb0)
        pltpu.sync_copy(rows0, acc_sh.at[dstb0], add=True)
        issue_idx(jnp.minimum(c0 + 2, NCH_M - 1), srcb0, dstb0, eeb0, isem0)
        pltpu.make_async_copy(h_hbm.at[srcb1], rows1, gsem1).wait()
        _msg_scale(rows1, eeb1)
        pltpu.sync_copy(rows1, acc_sh.at[dstb1], add=True)
        wait_idx(srcb0, dstb0, eeb0, isem0)
        pltpu.async_copy(h_hbm.at[srcb0], rows0, gsem0)
        issue_idx(jnp.minimum(c0 + 3, NCH_M - 1), srcb1, dstb1, eeb1, isem1)
        return carry

    lax.fori_loop(0, NCH_M // 2, pair_body, 0)

    pltpu.make_async_copy(h_hbm.at[srcb0], rows0, gsem0).wait()
    wait_idx(srcb1, dstb1, eeb1, isem1)

    plsc.subcore_barrier()

    pltpu.sync_copy(acc_sh.at[pl.ds(r0, ROWS_PT)], acc_out.at[cid, pl.ds(r0, ROWS_PT)])

    @pl.when(sid == NS - 1)
    def _():
        pltpu.sync_copy(acc_sh.at[pl.ds(TAIL0, TAILN)], acc_out.at[cid, pl.ds(TAIL0, TAILN)])


_sc_msg_pass = pl.kernel(
    _sc_msg_body,
    out_type=jax.ShapeDtypeStruct((NC, N, D), jnp.float32),
    mesh=_MESH,
    scratch_types=[
        pltpu.VMEM((KM,), jnp.int32),
        pltpu.VMEM((KM,), jnp.int32),
        pltpu.VMEM((KM * H,), jnp.float32),
        pltpu.VMEM((KM,), jnp.int32),
        pltpu.VMEM((KM,), jnp.int32),
        pltpu.VMEM((KM * H,), jnp.float32),
        pltpu.VMEM((KM, D), jnp.float32),
        pltpu.VMEM((KM, D), jnp.float32),
        pltpu.VMEM_SHARED((N, D), jnp.float32),
        pltpu.SemaphoreType.DMA,
        pltpu.SemaphoreType.DMA,
        pltpu.SemaphoreType.DMA,
        pltpu.SemaphoreType.DMA,
    ],
    compiler_params=pltpu.CompilerParams(needs_layout_passes=False),
)


KL = 128
NCH_L = EPT // KL
LTAIL = EPT - NCH_L * KL


def _sc_lp_body(src_hbm, dst_hbm, g_hbm, z128_hbm, agg_out,
                srcb0, dstb0, srcb1, dstb1, rows0, rows1, srct, dstt, rowst,
                acc_sh, gsem0, gsem1, isem0, isem1, tsem):
    cid = lax.axis_index("c")
    sid = lax.axis_index("s")
    wid = sid * NC + cid

    r0 = sid * ROWS_PT
    pltpu.sync_copy(z128_hbm.at[pl.ds(r0, ROWS_PT)], acc_sh.at[pl.ds(r0, ROWS_PT)])

    @pl.when(sid == NS - 1)
    def _():
        pltpu.sync_copy(z128_hbm.at[pl.ds(TAIL0, TAILN)], acc_sh.at[pl.ds(TAIL0, TAILN)])

    plsc.subcore_barrier()

    ebase = wid * EPT

    def issue_idx(c, sb, db, sem):
        base = pl.multiple_of(ebase + c * KL, 8)
        pltpu.async_copy(src_hbm.at[pl.ds(base, KL)], sb, sem)
        pltpu.async_copy(dst_hbm.at[pl.ds(base, KL)], db, sem)

    def wait_idx(sb, db, sem):
        pltpu.make_async_copy(src_hbm.at[pl.ds(0, KL)], sb, sem).wait()
        pltpu.make_async_copy(dst_hbm.at[pl.ds(0, KL)], db, sem).wait()

    issue_idx(0, srcb0, dstb0, isem0)
    wait_idx(srcb0, dstb0, isem0)
    pltpu.async_copy(g_hbm.at[srcb0], rows0, gsem0)
    issue_idx(1, srcb1, dstb1, isem1)

    def pair_body(cc, carry):
        c0 = cc * 2
        pltpu.make_async_copy(g_hbm.at[srcb0], rows0, gsem0).wait()
        wait_idx(srcb1, dstb1, isem1)
        pltpu.async_copy(g_hbm.at[srcb1], rows1, gsem1)
        pltpu.sync_copy(rows0, acc_sh.at[dstb0], add=True)
        issue_idx(jnp.minimum(c0 + 2, NCH_L - 1), srcb0, dstb0, isem0)
        pltpu.make_async_copy(g_hbm.at[srcb1], rows1, gsem1).wait()
        pltpu.sync_copy(rows1, acc_sh.at[dstb1], add=True)
        wait_idx(srcb0, dstb0, isem0)
        pltpu.async_copy(g_hbm.at[srcb0], rows0, gsem0)
        issue_idx(jnp.minimum(c0 + 3, NCH_L - 1), srcb1, dstb1, isem1)
        return carry

    lax.fori_loop(0, NCH_L // 2, pair_body, 0)

    pltpu.make_async_copy(g_hbm.at[srcb0], rows0, gsem0).wait()
    wait_idx(srcb1, dstb1, isem1)

    tbase = pl.multiple_of(ebase + NCH_L * KL, 8)
    pltpu.sync_copy(src_hbm.at[pl.ds(tbase, LTAIL)], srct)
    pltpu.sync_copy(dst_hbm.at[pl.ds(tbase, LTAIL)], dstt)
    pltpu.async_copy(g_hbm.at[srct], rowst, tsem).wait()
    pltpu.sync_copy(rowst, acc_sh.at[dstt], add=True)

    plsc.subcore_barrier()

    pltpu.sync_copy(acc_sh.at[pl.ds(r0, ROWS_PT)], agg_out.at[cid, pl.ds(r0, ROWS_PT)])

    @pl.when(sid == NS - 1)
    def _():
        pltpu.sync_copy(acc_sh.at[pl.ds(TAIL0, TAILN)], agg_out.at[cid, pl.ds(TAIL0, TAILN)])


_sc_lp_pass = pl.kernel(
    _sc_lp_body,
    out_type=jax.ShapeDtypeStruct((NC, N, D), jnp.float32),
    mesh=_MESH,
    scratch_types=[
        pltpu.VMEM((KL,), jnp.int32),
        pltpu.VMEM((KL,), jnp.int32),
        pltpu.VMEM((KL,), jnp.int32),
        pltpu.VMEM((KL,), jnp.int32),
        pltpu.VMEM((KL, D), jnp.float32),
        pltpu.VMEM((KL, D), jnp.float32),
        pltpu.VMEM((LTAIL,), jnp.int32),
        pltpu.VMEM((LTAIL,), jnp.int32),
        pltpu.VMEM((LTAIL, D), jnp.float32),
        pltpu.VMEM_SHARED((N, D), jnp.float32),
        pltpu.SemaphoreType.DMA,
        pltpu.SemaphoreType.DMA,
        pltpu.SemaphoreType.DMA,
        pltpu.SemaphoreType.DMA,
        pltpu.SemaphoreType.DMA,
    ],
    compiler_params=pltpu.CompilerParams(needs_layout_passes=False),
)


def _tc_proj_body(x_ref, w_ref, as_ref, ad_ref, h_ref, als_ref, ald_ref):
    h = jnp.dot(x_ref[...], w_ref[...], preferred_element_type=jnp.float32)
    h_ref[...] = h
    als_ref[...] = jnp.dot(h, as_ref[...], preferred_element_type=jnp.float32)
    ald_ref[...] = jnp.dot(h, ad_ref[...], preferred_element_type=jnp.float32)


def _tc_proj(x, w, a_s, a_d):
    return pl.pallas_call(
        _tc_proj_body,
        out_shape=(
            jax.ShapeDtypeStruct((N, D), jnp.float32),
            jax.ShapeDtypeStruct((N, H), jnp.float32),
            jax.ShapeDtypeStruct((N, H), jnp.float32),
        ),
    )(x, w, a_s, a_d)


def _dis_from_deg(degp):
    deg = degp[0] + degp[1]
    return jnp.where(deg > 0, lax.rsqrt(jnp.maximum(deg, 1e-12)), 0.0)


def _tc_gatfin_body(accp_ref, denp_ref, degp_ref, b_ref, s_ref, h_ref, g_ref):
    acc = accp_ref[0] + accp_ref[1]
    den = denp_ref[0] + denp_ref[1]
    r = 1.0 / (den + 1e-16)
    r_rep = jnp.dot(r, s_ref[...], preferred_element_type=jnp.float32)
    h = jnp.maximum(acc * r_rep + b_ref[...], 0.0)
    h_ref[...] = h
    g_ref[...] = h * _dis_from_deg(degp_ref)


def _tc_gatfin(accp, denp, degp, b2d, sel):
    return pl.pallas_call(
        _tc_gatfin_body,
        out_shape=(
            jax.ShapeDtypeStruct((N, D), jnp.float32),
            jax.ShapeDtypeStruct((N, D), jnp.float32),
        ),
    )(accp, denp, degp, b2d, sel)


def _tc_lpfin_body(aggp_ref, hres_ref, degp_ref, out_ref, g_ref):
    dis = _dis_from_deg(degp_ref)
    agg = aggp_ref[0] + aggp_ref[1]
    out = jnp.clip(0.5 * dis * agg + 0.5 * hres_ref[...], 0.0, 1.0)
    out_ref[...] = out
    g_ref[...] = out * dis


def _tc_lpfin(aggp, hres, degp):
    return pl.pallas_call(
        _tc_lpfin_body,
        out_shape=(
            jax.ShapeDtypeStruct((N, D), jnp.float32),
            jax.ShapeDtypeStruct((N, D), jnp.float32),
        ),
    )(aggp, hres, degp)


def _tc_final_body(x_ref, h1_ref, h2_ref, bt_ref, w1_ref, b1_ref, w2_ref, b2_ref,
                   out_ref):
    combined = jnp.concatenate([x_ref[...], h1_ref[...], h2_ref[...]], axis=-1)
    bt = bt_ref[...]
    oh = (lax.broadcasted_iota(jnp.int32, (B, N), 0) == bt).astype(jnp.float32)
    pooled_sum = jnp.dot(oh, combined, preferred_element_type=jnp.float32)
    counts = jnp.sum(oh, axis=1, keepdims=True)
    pooled = pooled_sum / jnp.maximum(counts, 1.0)
    hmid = jnp.maximum(
        jnp.dot(pooled, w1_ref[...], preferred_element_type=jnp.float32) + b1_ref[...],
        0.0)
    out_ref[...] = jnp.dot(hmid, w2_ref[...], preferred_element_type=jnp.float32) + b2_ref[...]


def _tc_final(x, h1, h2, bt, w1, b1, w2, b2):
    return pl.pallas_call(
        _tc_final_body,
        out_shape=jax.ShapeDtypeStruct((B, 128), jnp.float32),
    )(x, h1, h2, bt, w1, b1, w2, b2)


def kernel(x, edge_index, batch, W1, a1_src, a1_dst, b1, W2, a2_src, a2_dst, b2,
           mlp_w1, mlp_b1, mlp_w2, mlp_b2):
    src = edge_index[0]
    dst = edge_index[1]

    eye = jnp.eye(H, dtype=jnp.float32)
    As1 = jnp.einsum('hc,hk->hck', a1_src, eye).reshape(D, H)
    Ad1 = jnp.einsum('hc,hk->hck', a1_dst, eye).reshape(D, H)
    As2 = jnp.einsum('hc,hk->hck', a2_src, eye).reshape(D, H)
    Ad2 = jnp.einsum('hc,hk->hck', a2_dst, eye).reshape(D, H)
    sel = jnp.repeat(jnp.eye(H, dtype=jnp.float32), C, axis=1)

    z128 = jnp.zeros((N, D), jnp.float32)
    z4 = jnp.zeros((DENW,), jnp.float32)
    z1 = jnp.zeros((DEGW,), jnp.float32)
    b1_2d = b1.reshape(1, D)
    b2_2d = b2.reshape(1, D)
    bt = batch.reshape(1, N)

    h1p, als1, ald1 = _tc_proj(x, W1, As1, Ad1)
    denp1, degp1, ee1 = _sc_attn_pass(src, dst, als1.reshape(-1), ald1.reshape(-1), z4, z1)
    denp1 = denp1[:, 0, :N * H].reshape(NC, N, H)
    degp = degp1[:, 0, :N].reshape(NC, N, 1)
    accp1 = _sc_msg_pass(src, dst, h1p, ee1, z128)
    h1, g = _tc_gatfin(accp1, denp1, degp, b1_2d, sel)
    aggp = _sc_lp_pass(src, dst, g, z128)
    _, g = _tc_lpfin(aggp, h1, degp)
    aggp = _sc_lp_pass(src, dst, g, z128)
    h1f, _ = _tc_lpfin(aggp, h1, degp)

    h2p, als2, ald2 = _tc_proj(h1f, W2, As2, Ad2)
    denp2, _, ee2 = _sc_attn_pass(src, dst, als2.reshape(-1), ald2.reshape(-1), z4, z1)
    denp2 = denp2[:, 0, :N * H].reshape(NC, N, H)
    accp2 = _sc_msg_pass(src, dst, h2p, ee2, z128)
    h2, g = _tc_gatfin(accp2, denp2, degp, b2_2d, sel)
    aggp = _sc_lp_pass(src, dst, g, z128)
    _, g = _tc_lpfin(aggp, h2, degp)
    aggp = _sc_lp_pass(src, dst, g, z128)
    h2f, _ = _tc_lpfin(aggp, h2, degp)

    return _tc_final(x, h1f, h2f, bt, mlp_w1, mlp_b1.reshape(1, 256),
                     mlp_w2, mlp_b2.reshape(1, 128))
